# Initial kernel scaffold; baseline (speedup 1.0000x reference)
import jax
import jax.numpy as jnp
from jax.experimental import pallas as pl


def kernel(query_length, key_length, relative_attention_bias):
    raise NotImplementedError("write your pallas kernel here")


# speedup vs baseline: 15.6356x; 15.6356x over previous
import math

import jax
import jax.numpy as jnp
from jax.experimental import pallas as pl
from jax.experimental.pallas import tpu as pltpu

H = 16
NBUC = 32
QL = 2048
KL = 2048
W = 4096
BLK = 256


def _body(delta_ref, table_t_ref, out_ref, vals_ref):
    h = pl.program_id(0)
    nb = pl.program_id(1)

    @pl.when((h == 0) & (nb == 0))
    def _compute_vals():
        x = jax.lax.broadcasted_iota(jnp.int32, (H, W), 1)
        d = x - (QL - 1) + delta_ref[0]
        rb = jnp.where(d > 0, 16, 0).astype(jnp.int32)
        a = jnp.abs(d)
        is_small = a < 8
        rp_safe = jnp.maximum(a, 1)
        large = 8 + (
            jnp.log(rp_safe.astype(jnp.float32) / 8)
            / math.log(128 / 8)
            * (16 - 8)
        ).astype(jnp.int32)
        large = jnp.minimum(large, jnp.full_like(large, 15))
        bucket = rb + jnp.where(is_small, a, large)
        acc = jnp.zeros((H, W), jnp.float32)
        for b in range(NBUC):
            acc = jnp.where(bucket == b, table_t_ref[:, pl.ds(b, 1)], acc)
        vals_ref[...] = acc

    i0 = nb * BLK
    v = vals_ref[pl.ds(h, 1), :]

    for g in range(BLK // 8):
        base = (QL - 1) - (i0 + 8 * g + 7)
        rolled = pltpu.roll(v, W - base, axis=1)
        blk = jnp.broadcast_to(rolled[:, : KL + 128], (8, KL + 128))
        t = 7 - jax.lax.broadcasted_iota(jnp.int32, (8, KL + 128), 0)
        for k in range(3):
            sh = 1 << k
            shifted = pltpu.roll(blk, (KL + 128) - sh, axis=1)
            blk = jnp.where(((t >> k) & 1) == 1, shifted, blk)
        out_ref[0, 0, pl.ds(8 * g, 8), :] = blk[:, :KL]


def kernel(query_length, key_length, relative_attention_bias):
    delta = (
        (jnp.asarray(key_length, jnp.int32) - KL)
        - (jnp.asarray(query_length, jnp.int32) - QL)
    ).reshape(1)
    table_t = relative_attention_bias.T
    out = pl.pallas_call(
        _body,
        grid=(H, QL // BLK),
        in_specs=[
            pl.BlockSpec(memory_space=pltpu.SMEM),
            pl.BlockSpec((H, NBUC), lambda h, nb: (0, 0)),
        ],
        out_specs=pl.BlockSpec((1, 1, BLK, KL), lambda h, nb: (0, h, nb, 0)),
        out_shape=jax.ShapeDtypeStruct((1, H, QL, KL), jnp.float32),
        scratch_shapes=[pltpu.VMEM((H, W), jnp.float32)],
    )(delta, table_t)
    return out


# speedup vs baseline: 38.3415x; 2.4522x over previous
import math

import jax
import jax.numpy as jnp
from jax.experimental import pallas as pl
from jax.experimental.pallas import tpu as pltpu

H = 16
NBUC = 32
QL = 2048
KL = 2048
QH = 40
RB = 128


def _flat_shift(x, n):
    rl = pltpu.roll(x, 128 - n, axis=2)
    sub = pltpu.roll(rl, QH - 1, axis=1)
    lane = jax.lax.broadcasted_iota(jnp.int32, x.shape, 2)
    return jnp.where(lane < 128 - n, rl, sub)


def _body(delta_ref, table_t_ref, out_ref, vals_ref, bank_ref):
    h = pl.program_id(0)
    g = pl.program_id(1)

    @pl.when((h == 0) & (g == 0))
    def _compute_vals():
        q = jax.lax.broadcasted_iota(jnp.int32, (H, QH, 128), 1)
        l = jax.lax.broadcasted_iota(jnp.int32, (H, QH, 128), 2)
        d = 128 * q + l - (QL - 1) + delta_ref[0]
        rb = jnp.where(d > 0, 16, 0).astype(jnp.int32)
        a = jnp.abs(d)
        is_small = a < 8
        rp_safe = jnp.maximum(a, 1)
        large = 8 + (
            jnp.log(rp_safe.astype(jnp.float32) / 8)
            / math.log(128 / 8)
            * (16 - 8)
        ).astype(jnp.int32)
        large = jnp.minimum(large, jnp.full_like(large, 15))
        bucket = rb + jnp.where(is_small, a, large)
        acc = jnp.zeros((H, QH, 128), jnp.float32)
        for b in range(NBUC):
            acc = jnp.where(bucket == b, table_t_ref[:, pl.ds(b, 1)][:, :, None], acc)
        vals_ref[...] = acc

    @pl.when(g == 0)
    def _build_bank():
        bank_ref[RB - 1, :, :] = vals_ref[h]
        for k in range(7):
            n = 1 << k
            src = bank_ref[RB - n : RB, :, :]
            bank_ref[RB - 2 * n : RB - n, :, :] = _flat_shift(src, n)

    out_ref[0] = bank_ref[:, pl.ds(15 - g, 16), :]


def kernel(query_length, key_length, relative_attention_bias):
    delta = (
        (jnp.asarray(key_length, jnp.int32) - KL)
        - (jnp.asarray(query_length, jnp.int32) - QL)
    ).reshape(1)
    table_t = relative_attention_bias.T
    out = pl.pallas_call(
        _body,
        grid=(H, QL // RB),
        in_specs=[
            pl.BlockSpec(memory_space=pltpu.SMEM),
            pl.BlockSpec((H, NBUC), lambda h, g: (0, 0)),
        ],
        out_specs=pl.BlockSpec((1, RB, 16, 128), lambda h, g: (h, g, 0, 0)),
        out_shape=jax.ShapeDtypeStruct((H, QL, 16, 128), jnp.float32),
        scratch_shapes=[
            pltpu.VMEM((H, QH, 128), jnp.float32),
            pltpu.VMEM((RB, QH, 128), jnp.float32),
        ],
    )(delta, table_t)
    return out.reshape(1, H, QL, KL)


# speedup vs baseline: 42.1341x; 1.0989x over previous
import functools
import math

import jax
import jax.numpy as jnp
from jax import lax
from jax.experimental import pallas as pl
from jax.experimental.pallas import tpu as pltpu
from jax.experimental.pallas import tpu_sc as plsc

H = 16
NBUC = 32
QL = 2048
KL = 2048
VW = 4224
NSHIFT = 8
WAVE = 8


def _vals_body(delta_ref, table_t_ref, vals8_ref):
    r = jax.lax.broadcasted_iota(jnp.int32, (H, NSHIFT, VW), 1)
    x = jax.lax.broadcasted_iota(jnp.int32, (H, NSHIFT, VW), 2)
    d = x - r - (QL - 1) + delta_ref[0]
    rb = jnp.where(d > 0, 16, 0).astype(jnp.int32)
    a = jnp.abs(d)
    is_small = a < 8
    rp_safe = jnp.maximum(a, 1)
    large = 8 + (
        jnp.log(rp_safe.astype(jnp.float32) / 8)
        / math.log(128 / 8)
        * (16 - 8)
    ).astype(jnp.int32)
    large = jnp.minimum(large, jnp.full_like(large, 15))
    bucket = rb + jnp.where(is_small, a, large)
    acc = jnp.zeros((H, NSHIFT, VW), jnp.float32)
    for b in range(NBUC):
        acc = jnp.where(bucket == b, table_t_ref[:, pl.ds(b, 1)][:, :, None], acc)
    vals8_ref[...] = acc


def _sc_body(vals8_hbm, out_hbm, vv, sem):
    wid = lax.axis_index("s") * 2 + lax.axis_index("c")
    head = wid // 2
    base = (wid % 2) * (QL // 2)
    pltpu.sync_copy(vals8_hbm.at[pl.ds(head * (NSHIFT * VW), NSHIFT * VW)], vv)

    def wave(w, carry):
        copies = []
        for u in range(WAVE):
            i = base + w * WAVE + u
            s = (QL - 1) - i
            r = (i + 1) % NSHIFT
            off = pl.multiple_of(r * VW + s + r, 8)
            dst = pl.multiple_of((head * QL + i) * KL, 8)
            copies.append(
                pltpu.async_copy(vv.at[pl.ds(off, KL)], out_hbm.at[pl.ds(dst, KL)], sem)
            )
        for cp in copies:
            cp.wait()
        return carry

    lax.fori_loop(0, (QL // 2) // WAVE, wave, 0, unroll=False)


def kernel(query_length, key_length, relative_attention_bias):
    delta = (
        (jnp.asarray(key_length, jnp.int32) - KL)
        - (jnp.asarray(query_length, jnp.int32) - QL)
    ).reshape(1)
    table_t = relative_attention_bias.T
    vals8 = pl.pallas_call(
        _vals_body,
        in_specs=[
            pl.BlockSpec(memory_space=pltpu.SMEM),
            pl.BlockSpec((H, NBUC), lambda: (0, 0)),
        ],
        out_specs=pl.BlockSpec((H, NSHIFT, VW), lambda: (0, 0, 0)),
        out_shape=jax.ShapeDtypeStruct((H, NSHIFT, VW), jnp.float32),
    )(delta, table_t)

    sc_call = functools.partial(
        pl.kernel,
        out_type=jax.ShapeDtypeStruct((H * QL * KL,), jnp.float32),
        mesh=plsc.VectorSubcoreMesh(core_axis_name="c", subcore_axis_name="s"),
        scratch_types=[
            pltpu.VMEM((NSHIFT * VW,), jnp.float32),
            pltpu.SemaphoreType.DMA,
        ],
    )(_sc_body)
    out = sc_call(vals8.reshape(H * NSHIFT * VW))
    return out.reshape(1, H, QL, KL)
